# Initial kernel scaffold; baseline (speedup 1.0000x reference)
import jax
import jax.numpy as jnp
from jax.experimental import pallas as pl


def kernel(xyz, W1, W2):
    raise NotImplementedError("write your pallas kernel here")


# speedup vs baseline: 4.2262x; 4.2262x over previous
import functools

import jax
import jax.numpy as jnp
from jax.experimental import pallas as pl

K = 16
N_COUT = 64
QBLK = 256


def _knn_conv_kernel(xyz_ref, xyzT8_ref, w1p_ref, w2_ref, out_ref):
    qi = pl.program_id(1)
    pts = xyz_ref[0, :, :]
    ptsT8 = xyzT8_ref[0, :, :]
    n = pts.shape[0]

    qT = xyzT8_ref[0, :, pl.ds(qi * QBLK, QBLK)]
    dx = pts[:, 0:1] - qT[0:1, :]
    dy = pts[:, 1:2] - qT[1:2, :]
    dz = pts[:, 2:3] - qT[2:3, :]
    dist = (dx * dx + dy * dy) + dz * dz

    row_iota = jax.lax.broadcasted_iota(jnp.int32, (n, QBLK), 0)
    col_gidx = jax.lax.broadcasted_iota(jnp.int32, (n, QBLK), 1) + qi * QBLK
    inf = jnp.float32(jnp.inf)
    dist = jnp.where(row_iota == col_gidx, inf, dist)

    w1p = w1p_ref[...]
    hi = jax.lax.Precision.HIGHEST
    pwT = jnp.dot(w1p, ptsT8, preferred_element_type=jnp.float32,
                  precision=hi)
    w1q = jnp.dot(w1p, qT, preferred_element_type=jnp.float32,
                  precision=hi)
    w2 = w2_ref[...]

    acc = jnp.full((N_COUT, QBLK), -inf, dtype=jnp.float32)
    for _ in range(K):
        m = jnp.min(dist, axis=0, keepdims=True)
        cand = jnp.where(dist == m, row_iota, n)
        jsel = jnp.min(cand, axis=0, keepdims=True)
        onehot = (row_iota == jsel).astype(jnp.float32)
        dist = jnp.where(row_iota == jsel, inf, dist)
        g = jnp.dot(pwT, onehot, preferred_element_type=jnp.float32,
                    precision=hi)
        h1 = jnp.maximum(g - w1q, 0.0)
        h2 = jnp.dot(w2, h1, preferred_element_type=jnp.float32,
                     precision=hi)
        acc = jnp.maximum(acc, h2)

    out_ref[0, :, :] = acc


@jax.jit
def kernel(xyz, W1, W2):
    B, N, _ = xyz.shape
    xyzT8 = jnp.zeros((B, 8, N), jnp.float32).at[:, :3, :].set(
        jnp.transpose(xyz, (0, 2, 1)))
    w1p = jnp.zeros((N_COUT, 8), jnp.float32).at[:, :3].set(W1)

    grid = (B, N // QBLK)
    return pl.pallas_call(
        _knn_conv_kernel,
        grid=grid,
        in_specs=[
            pl.BlockSpec((1, N, 3), lambda b, q: (b, 0, 0)),
            pl.BlockSpec((1, 8, N), lambda b, q: (b, 0, 0)),
            pl.BlockSpec((N_COUT, 8), lambda b, q: (0, 0)),
            pl.BlockSpec((N_COUT, N_COUT), lambda b, q: (0, 0)),
        ],
        out_specs=pl.BlockSpec((1, N_COUT, QBLK), lambda b, q: (b, 0, q)),
        out_shape=jax.ShapeDtypeStruct((B, N_COUT, N), jnp.float32),
    )(xyz, xyzT8, w1p, W2)


# speedup vs baseline: 9.0930x; 2.1516x over previous
import functools

import jax
import jax.numpy as jnp
from jax.experimental import pallas as pl

K = 16
N_COUT = 64
QBLK = 256


def _knn_conv_kernel(xyz_ref, xyzT8_ref, w1p_ref, w2_ref, out_ref):
    qi = pl.program_id(1)
    pts = xyz_ref[0, :, :]
    ptsT8 = xyzT8_ref[0, :, :]
    n = pts.shape[0]

    qT = xyzT8_ref[0, :, pl.ds(qi * QBLK, QBLK)]
    dx = pts[:, 0:1] - qT[0:1, :]
    dy = pts[:, 1:2] - qT[1:2, :]
    dz = pts[:, 2:3] - qT[2:3, :]
    dist = (dx * dx + dy * dy) + dz * dz

    row_iota = jax.lax.broadcasted_iota(jnp.int32, (n, QBLK), 0)
    col_gidx = jax.lax.broadcasted_iota(jnp.int32, (n, QBLK), 1) + qi * QBLK
    inf = jnp.float32(jnp.inf)
    dist = jnp.where(row_iota == col_gidx, inf, dist)

    w1p = w1p_ref[...]
    hi = jax.lax.Precision.HIGHEST
    pwT = jnp.dot(w1p, ptsT8, preferred_element_type=jnp.float32,
                  precision=hi)
    w1q = jnp.dot(w1p, qT, preferred_element_type=jnp.float32,
                  precision=hi)
    w2 = w2_ref[...]
    pw_hi = pwT.astype(jnp.bfloat16)
    pw_lo = (pwT - pw_hi.astype(jnp.float32)).astype(jnp.bfloat16)

    acc = jnp.full((N_COUT, QBLK), -inf, dtype=jnp.float32)
    for _ in range(K):
        m = jnp.min(dist, axis=0, keepdims=True)
        cand = jnp.where(dist == m, row_iota, n)
        jsel = jnp.min(cand, axis=0, keepdims=True)
        selmask = row_iota == jsel
        onehot = jnp.where(selmask, 1.0, 0.0).astype(jnp.bfloat16)
        dist = jnp.where(selmask, inf, dist)
        g = (jnp.dot(pw_hi, onehot, preferred_element_type=jnp.float32)
             + jnp.dot(pw_lo, onehot, preferred_element_type=jnp.float32))
        h1 = jnp.maximum(g - w1q, 0.0)
        h2 = jnp.dot(w2, h1, preferred_element_type=jnp.float32,
                     precision=hi)
        acc = jnp.maximum(acc, h2)

    out_ref[0, :, :] = acc


@jax.jit
def kernel(xyz, W1, W2):
    B, N, _ = xyz.shape
    xyzT8 = jnp.zeros((B, 8, N), jnp.float32).at[:, :3, :].set(
        jnp.transpose(xyz, (0, 2, 1)))
    w1p = jnp.zeros((N_COUT, 8), jnp.float32).at[:, :3].set(W1)

    grid = (B, N // QBLK)
    return pl.pallas_call(
        _knn_conv_kernel,
        grid=grid,
        in_specs=[
            pl.BlockSpec((1, N, 3), lambda b, q: (b, 0, 0)),
            pl.BlockSpec((1, 8, N), lambda b, q: (b, 0, 0)),
            pl.BlockSpec((N_COUT, 8), lambda b, q: (0, 0)),
            pl.BlockSpec((N_COUT, N_COUT), lambda b, q: (0, 0)),
        ],
        out_specs=pl.BlockSpec((1, N_COUT, QBLK), lambda b, q: (b, 0, q)),
        out_shape=jax.ShapeDtypeStruct((B, N_COUT, N), jnp.float32),
    )(xyz, xyzT8, w1p, W2)


# speedup vs baseline: 9.8725x; 1.0857x over previous
import functools

import jax
import jax.numpy as jnp
from jax import lax
from jax.experimental import pallas as pl
from jax.experimental.pallas import tpu as pltpu
from jax.experimental.pallas import tpu_sc as plsc

K = 16
N_COUT = 64
QBLK = 256
_HI = jax.lax.Precision.HIGHEST


def _select_kernel(xyz_ref, xyzT8_ref, w1t_ref, idx_ref, pw_ref):
    b = pl.program_id(0)
    qi = pl.program_id(1)
    pts = xyz_ref[0, :, :]
    n = pts.shape[0]

    @pl.when(qi == 0)
    def _():
        pw_ref[0, :, :] = jnp.dot(pts, w1t_ref[...], precision=_HI)

    qT = xyzT8_ref[0, :, pl.ds(qi * QBLK, QBLK)]
    dx = pts[:, 0:1] - qT[0:1, :]
    dy = pts[:, 1:2] - qT[1:2, :]
    dz = pts[:, 2:3] - qT[2:3, :]
    dist = (dx * dx + dy * dy) + dz * dz

    row_iota = jax.lax.broadcasted_iota(jnp.int32, (n, QBLK), 0)
    col_gidx = jax.lax.broadcasted_iota(jnp.int32, (n, QBLK), 1) + qi * QBLK
    inf = jnp.float32(jnp.inf)
    dist = jnp.where(row_iota == col_gidx, inf, dist)

    rows = []
    for _ in range(K):
        m = jnp.min(dist, axis=0, keepdims=True)
        cand = jnp.where(dist == m, row_iota, n)
        jsel = jnp.min(cand, axis=0, keepdims=True)
        dist = jnp.where(row_iota == jsel, inf, dist)
        rows.append(jsel + b * n)
    idx_ref[0, :, :] = jnp.concatenate(rows, axis=0)


def _conv_kernel(g_ref, xyzq_ref, w1t_ref, w2t_ref, out_ref):
    qpts = xyzq_ref[0, :, :]
    w1q = jnp.dot(qpts, w1t_ref[...], precision=_HI)
    w2t = w2t_ref[...]
    w2h = w2t.astype(jnp.bfloat16)
    w2l = (w2t - w2h.astype(jnp.float32)).astype(jnp.bfloat16)

    acc = jnp.full((QBLK, N_COUT), -jnp.inf, dtype=jnp.float32)
    for k in range(K):
        h1 = jnp.maximum(g_ref[0, k, :, :N_COUT] - w1q, 0.0)
        h1h = h1.astype(jnp.bfloat16)
        h1l = (h1 - h1h.astype(jnp.float32)).astype(jnp.bfloat16)
        h2 = (jnp.dot(h1h, w2h, preferred_element_type=jnp.float32)
              + (jnp.dot(h1h, w2l, preferred_element_type=jnp.float32)
                 + jnp.dot(h1l, w2h, preferred_element_type=jnp.float32)))
        acc = jnp.maximum(acc, h2)
    out_ref[0, :, :] = acc


def _sc_gather_body(table_hbm, idx_hbm, out_hbm, idx_v, rows_v, sem):
    c = lax.axis_index("c")
    s = lax.axis_index("s")
    wid = s * 2 + c
    rows_per_worker = 4096
    chunk = 512
    for ch in range(rows_per_worker // chunk):
        base = wid * rows_per_worker + ch * chunk
        pltpu.sync_copy(idx_hbm.at[pl.ds(base, chunk)], idx_v)
        pltpu.async_copy(table_hbm.at[idx_v], rows_v, sem).wait()
        pltpu.sync_copy(rows_v, out_hbm.at[pl.ds(base, chunk)])


@jax.jit
def kernel(xyz, W1, W2):
    B, N, _ = xyz.shape
    xyzT8 = jnp.zeros((B, 8, N), jnp.float32).at[:, :3, :].set(
        jnp.transpose(xyz, (0, 2, 1)))
    w1t = jnp.transpose(W1, (1, 0))
    w2t = jnp.transpose(W2, (1, 0))

    grid = (B, N // QBLK)
    idx, pw = pl.pallas_call(
        _select_kernel,
        grid=grid,
        in_specs=[
            pl.BlockSpec((1, N, 3), lambda b, q: (b, 0, 0)),
            pl.BlockSpec((1, 8, N), lambda b, q: (b, 0, 0)),
            pl.BlockSpec((3, N_COUT), lambda b, q: (0, 0)),
        ],
        out_specs=[
            pl.BlockSpec((1, K, QBLK), lambda b, q: (b, 0, q)),
            pl.BlockSpec((1, N, N_COUT), lambda b, q: (b, 0, 0)),
        ],
        out_shape=[
            jax.ShapeDtypeStruct((B, K, N), jnp.int32),
            jax.ShapeDtypeStruct((B, N, N_COUT), jnp.float32),
        ],
    )(xyz, xyzT8, w1t)

    idx_flat = idx.reshape(B * K * N)
    pw_pad = jnp.zeros((B * N, 128), jnp.float32).at[:, :N_COUT].set(
        pw.reshape(B * N, N_COUT))

    mesh = plsc.VectorSubcoreMesh(core_axis_name="c", subcore_axis_name="s")
    g_flat = pl.kernel(
        _sc_gather_body,
        out_type=jax.ShapeDtypeStruct((B * K * N, 128), jnp.float32),
        mesh=mesh,
        scratch_types=[
            pltpu.VMEM((512,), jnp.int32),
            pltpu.VMEM((512, 128), jnp.float32),
            pltpu.SemaphoreType.DMA,
        ],
    )(pw_pad, idx_flat)

    g4 = g_flat.reshape(B, K, N, 128)

    out_nc = pl.pallas_call(
        _conv_kernel,
        grid=grid,
        in_specs=[
            pl.BlockSpec((1, K, QBLK, 128), lambda b, q: (b, 0, q, 0)),
            pl.BlockSpec((1, QBLK, 3), lambda b, q: (b, q, 0)),
            pl.BlockSpec((3, N_COUT), lambda b, q: (0, 0)),
            pl.BlockSpec((N_COUT, N_COUT), lambda b, q: (0, 0)),
        ],
        out_specs=pl.BlockSpec((1, QBLK, N_COUT), lambda b, q: (b, q, 0)),
        out_shape=jax.ShapeDtypeStruct((B, N, N_COUT), jnp.float32),
    )(g4, xyz, w1t, w2t)

    return jnp.transpose(out_nc, (0, 2, 1))


# speedup vs baseline: 10.1660x; 1.0297x over previous
import functools

import jax
import jax.numpy as jnp
from jax import lax
from jax.experimental import pallas as pl
from jax.experimental.pallas import tpu as pltpu
from jax.experimental.pallas import tpu_sc as plsc

K = 16
N_COUT = 64
QBLK = 256
_HI = jax.lax.Precision.HIGHEST


def _select_kernel(xyz_ref, xyzT8_ref, w1t_ref, idx_ref, pw_ref):
    b = pl.program_id(0)
    qi = pl.program_id(1)
    pts = xyz_ref[0, :, :]
    n = pts.shape[0]

    @pl.when(qi == 0)
    def _():
        pw_ref[0, :, :] = jnp.dot(pts, w1t_ref[...], precision=_HI)

    qT = xyzT8_ref[0, :, pl.ds(qi * QBLK, QBLK)]
    dx = pts[:, 0:1] - qT[0:1, :]
    dy = pts[:, 1:2] - qT[1:2, :]
    dz = pts[:, 2:3] - qT[2:3, :]
    dist = (dx * dx + dy * dy) + dz * dz

    row_iota = jax.lax.broadcasted_iota(jnp.int32, (n, QBLK), 0)
    col_gidx = jax.lax.broadcasted_iota(jnp.int32, (n, QBLK), 1) + qi * QBLK
    inf = jnp.float32(jnp.inf)
    dist = jnp.where(row_iota == col_gidx, inf, dist)

    rows = []
    for _ in range(K):
        m = jnp.min(dist, axis=0, keepdims=True)
        cand = jnp.where(dist == m, row_iota, n)
        jsel = jnp.min(cand, axis=0, keepdims=True)
        dist = jnp.where(row_iota == jsel, inf, dist)
        rows.append(jsel + b * n)
    idx_ref[0, :, :] = jnp.concatenate(rows, axis=0)


def _conv_kernel(g_ref, xyzq_ref, w1t_ref, w2t_ref, out_ref):
    qpts = xyzq_ref[0, :, :]
    w1q = jnp.dot(qpts, w1t_ref[...], precision=_HI)
    w2t = w2t_ref[...]
    w2h = w2t.astype(jnp.bfloat16)
    w2l = (w2t - w2h.astype(jnp.float32)).astype(jnp.bfloat16)

    acc = jnp.full((QBLK, N_COUT), -jnp.inf, dtype=jnp.float32)
    for k in range(K):
        h1 = jnp.maximum(g_ref[0, k, :, :N_COUT] - w1q, 0.0)
        h1h = h1.astype(jnp.bfloat16)
        h1l = (h1 - h1h.astype(jnp.float32)).astype(jnp.bfloat16)
        h2 = (jnp.dot(h1h, w2h, preferred_element_type=jnp.float32)
              + (jnp.dot(h1h, w2l, preferred_element_type=jnp.float32)
                 + jnp.dot(h1l, w2h, preferred_element_type=jnp.float32)))
        acc = jnp.maximum(acc, h2)
    out_ref[0, :, :] = acc


def _sc_gather_body(table_hbm, idx_hbm, out_hbm, idx_v, rows_v, sem):
    c = lax.axis_index("c")
    s = lax.axis_index("s")
    wid = s * 2 + c
    rows_per_worker = 1024
    chunk = 512
    for ch in range(rows_per_worker // chunk):
        base = wid * rows_per_worker + ch * chunk
        pltpu.sync_copy(idx_hbm.at[pl.ds(base, chunk)], idx_v)
        pltpu.async_copy(table_hbm.at[idx_v], rows_v, sem).wait()
        pltpu.sync_copy(rows_v, out_hbm.at[pl.ds(base, chunk)])


@jax.jit
def kernel(xyz, W1, W2):
    B, N, _ = xyz.shape
    xyzT8 = jnp.zeros((B, 8, N), jnp.float32).at[:, :3, :].set(
        jnp.transpose(xyz, (0, 2, 1)))
    w1t = jnp.transpose(W1, (1, 0))
    w2t = jnp.transpose(W2, (1, 0))

    grid = (1, N // QBLK)
    mesh = plsc.VectorSubcoreMesh(core_axis_name="c", subcore_axis_name="s")

    outs = []
    for b in range(B):
        idx, pw = pl.pallas_call(
            _select_kernel,
            grid=grid,
            in_specs=[
                pl.BlockSpec((1, N, 3), lambda _, q: (0, 0, 0)),
                pl.BlockSpec((1, 8, N), lambda _, q: (0, 0, 0)),
                pl.BlockSpec((3, N_COUT), lambda _, q: (0, 0)),
            ],
            out_specs=[
                pl.BlockSpec((1, K, QBLK), lambda _, q: (0, 0, q)),
                pl.BlockSpec((1, N, N_COUT), lambda _, q: (0, 0, 0)),
            ],
            out_shape=[
                jax.ShapeDtypeStruct((1, K, N), jnp.int32),
                jax.ShapeDtypeStruct((1, N, N_COUT), jnp.float32),
            ],
        )(xyz[b:b + 1], xyzT8[b:b + 1], w1t)

        idx_flat = idx.reshape(K * N)
        pw_pad = jnp.zeros((N, 128), jnp.float32).at[:, :N_COUT].set(
            pw.reshape(N, N_COUT))

        g_flat = pl.kernel(
            _sc_gather_body,
            out_type=jax.ShapeDtypeStruct((K * N, 128), jnp.float32),
            mesh=mesh,
            scratch_types=[
                pltpu.VMEM((512,), jnp.int32),
                pltpu.VMEM((512, 128), jnp.float32),
                pltpu.SemaphoreType.DMA,
            ],
        )(pw_pad, idx_flat)

        g4 = g_flat.reshape(1, K, N, 128)

        out_nc = pl.pallas_call(
            _conv_kernel,
            grid=grid,
            in_specs=[
                pl.BlockSpec((1, K, QBLK, 128), lambda _, q: (0, 0, q, 0)),
                pl.BlockSpec((1, QBLK, 3), lambda _, q: (0, q, 0)),
                pl.BlockSpec((3, N_COUT), lambda _, q: (0, 0)),
                pl.BlockSpec((N_COUT, N_COUT), lambda _, q: (0, 0)),
            ],
            out_specs=pl.BlockSpec((1, QBLK, N_COUT), lambda _, q: (0, q, 0)),
            out_shape=jax.ShapeDtypeStruct((1, N, N_COUT), jnp.float32),
        )(g4, xyz[b:b + 1], w1t, w2t)
        outs.append(out_nc)

    return jnp.transpose(jnp.concatenate(outs, axis=0), (0, 2, 1))
